# Initial kernel scaffold; baseline (speedup 1.0000x reference)
import jax
import jax.numpy as jnp
from jax.experimental import pallas as pl


def kernel(x, edge_index, Ws, bs, gammas, betas):
    raise NotImplementedError("write your pallas kernel here")


# speedup vs baseline: 2.8230x; 2.8230x over previous
import functools

import jax
import jax.numpy as jnp
from jax import lax
from jax.experimental import pallas as pl
from jax.experimental.pallas import tpu as pltpu
from jax.experimental.pallas import tpu_sc as plsc

_N = 10000
_E = 320000
_D = 128
_L = 3
_EPS = 1e-5

_NC = 2
_NS = 16
_NW = _NC * _NS
_K = 128
_CH = 80
_EPT = _K * _CH
_E_PAD = _NW * _EPT
_NP = 10240
_RPT = _NP // _NS
_RC = _RPT // _K
_DW = 128


def _sc_mesh():
    return plsc.VectorSubcoreMesh(core_axis_name="c", subcore_axis_name="s")


def _sc_layer_body(h_hbm, src_hbm, dst_hbm, z128_hbm, iota_hbm, out_hbm,
                   src_v, dst_v, rows_v, msg, agg_sh, sem):
    c = lax.axis_index("c")
    s = lax.axis_index("s")
    wid = s * _NC + c

    pltpu.sync_copy(src_hbm.at[pl.ds(wid * _CH, _CH)], src_v)
    pltpu.sync_copy(dst_hbm.at[pl.ds(wid * _CH, _CH)], dst_v)
    pltpu.sync_copy(iota_hbm.at[s], rows_v)

    pltpu.sync_copy(z128_hbm, msg)
    for i in range(_RC):
        pltpu.sync_copy(msg, agg_sh.at[rows_v.at[i]])
    plsc.subcore_barrier()

    def step(j, carry):
        pltpu.async_copy(h_hbm.at[src_v.at[j]], msg, sem).wait()
        pltpu.sync_copy(msg, agg_sh.at[dst_v.at[j]], add=True)
        return carry

    lax.fori_loop(0, _CH, step, 0)
    plsc.subcore_barrier()

    for i in range(_RC):
        pltpu.sync_copy(agg_sh.at[rows_v.at[i]], msg)
        pltpu.sync_copy(
            msg, out_hbm.at[pl.ds(c * _NP + s * _RPT + i * _K, _K)])


def _make_sc_layer():
    return pl.kernel(
        _sc_layer_body,
        out_type=[jax.ShapeDtypeStruct((_NC * _NP, _D), jnp.float32)],
        mesh=_sc_mesh(),
        scratch_types=[
            pltpu.VMEM((_CH, _K), jnp.int32),
            pltpu.VMEM((_CH, _K), jnp.int32),
            pltpu.VMEM((_RC, _K), jnp.int32),
            pltpu.VMEM((_K, _D), jnp.float32),
            pltpu.VMEM_SHARED((_NP, _D), jnp.float32),
            pltpu.SemaphoreType.DMA,
        ],
    )


def _sc_deg_body(dst_hbm, zdeg_hbm, ones_hbm, iota_hbm, degout_hbm,
                 dst_v, rows_v, ones_v, degbuf, deg_sh):
    c = lax.axis_index("c")
    s = lax.axis_index("s")
    wid = s * _NC + c

    pltpu.sync_copy(dst_hbm.at[pl.ds(wid * _CH, _CH)], dst_v)
    pltpu.sync_copy(iota_hbm.at[s], rows_v)
    pltpu.sync_copy(ones_hbm, ones_v)
    pltpu.sync_copy(zdeg_hbm, degbuf)
    for i in range(_RC):
        pltpu.sync_copy(degbuf, deg_sh.at[rows_v.at[i]])
    plsc.subcore_barrier()

    def step(j, carry):
        pltpu.sync_copy(ones_v, deg_sh.at[dst_v.at[j]], add=True)
        return carry

    lax.fori_loop(0, _CH, step, 0)
    plsc.subcore_barrier()

    for i in range(_RC):
        pltpu.sync_copy(deg_sh.at[rows_v.at[i]], degbuf)
        pltpu.sync_copy(
            degbuf, degout_hbm.at[pl.ds(c * _NP + s * _RPT + i * _K, _K)])


def _make_sc_deg():
    return pl.kernel(
        _sc_deg_body,
        out_type=[jax.ShapeDtypeStruct((_NC * _NP, _DW), jnp.float32)],
        mesh=_sc_mesh(),
        scratch_types=[
            pltpu.VMEM((_CH, _K), jnp.int32),
            pltpu.VMEM((_RC, _K), jnp.int32),
            pltpu.VMEM((_K, _DW), jnp.float32),
            pltpu.VMEM((_K, _DW), jnp.float32),
            pltpu.VMEM_SHARED((_NP, _DW), jnp.float32),
        ],
    )


def _tc_layer_body(relu, p_ref, deg_ref, w_ref, g_ref, bt_ref, out_ref):
    p0 = p_ref[0:_NP, :]
    p1 = p_ref[_NP:2 * _NP, :]
    deg = deg_ref[0:_NP, 0:1] + deg_ref[_NP:2 * _NP, 0:1]
    rows = lax.broadcasted_iota(jnp.int32, (_NP, 1), 0)
    inv = jnp.where(rows < _N, 1.0 / jnp.maximum(deg, 1.0), 0.0)
    agg = (p0 + p1) * inv
    z = jnp.dot(agg, w_ref[...], preferred_element_type=jnp.float32)
    mean = jnp.sum(z, axis=0, keepdims=True) * (1.0 / _N)
    var = jnp.sum(z * z, axis=0, keepdims=True) * (1.0 / _N) - mean * mean
    h = (z - mean) * lax.rsqrt(var + _EPS) * g_ref[...] + bt_ref[...]
    if relu:
        h = jnp.maximum(h, 0.0)
    out_ref[...] = h[0:_N, :]


def _make_tc_layer(relu):
    return pl.pallas_call(
        functools.partial(_tc_layer_body, relu),
        out_shape=jax.ShapeDtypeStruct((_N, _D), jnp.float32),
    )


_sc_layer = _make_sc_layer()
_sc_deg = _make_sc_deg()
_tc_layer_relu = _make_tc_layer(True)
_tc_layer_last = _make_tc_layer(False)


def kernel(x, edge_index, Ws, bs, gammas, betas):
    del bs
    src = edge_index[0]
    dst = edge_index[1]
    pad = _E_PAD - _E
    src2 = jnp.concatenate([src, jnp.zeros((pad,), jnp.int32)]).reshape(
        _E_PAD // _K, _K)
    dst2 = jnp.concatenate([dst, jnp.full((pad,), _N, jnp.int32)]).reshape(
        _E_PAD // _K, _K)
    iota2 = jnp.arange(_NP, dtype=jnp.int32).reshape(_NS, _RC, _K)
    z128 = jnp.zeros((_K, _D), jnp.float32)
    zdeg = jnp.zeros((_K, _DW), jnp.float32)
    ones = jnp.ones((_K, _DW), jnp.float32)

    (degflat,) = _sc_deg(dst2, zdeg, ones, iota2)
    h = x
    for l in range(_L):
        (partials,) = _sc_layer(h, src2, dst2, z128, iota2)
        tc = _tc_layer_relu if l < _L - 1 else _tc_layer_last
        h = tc(partials, degflat, Ws[l],
               gammas[l].reshape(1, _D), betas[l].reshape(1, _D))
    return h


# speedup vs baseline: 3.0201x; 1.0698x over previous
import functools

import jax
import jax.numpy as jnp
from jax import lax
from jax.experimental import pallas as pl
from jax.experimental.pallas import tpu as pltpu
from jax.experimental.pallas import tpu_sc as plsc

_N = 10000
_E = 320000
_D = 128
_L = 3
_EPS = 1e-5

_NC = 2
_NS = 16
_NW = _NC * _NS
_K = 128
_CH = 80
_EPT = _K * _CH
_E_PAD = _NW * _EPT
_NP = 10240
_RPT = _NP // _NS
_RC = _RPT // _K
_DW = 128


def _sc_mesh():
    return plsc.VectorSubcoreMesh(core_axis_name="c", subcore_axis_name="s")


_NB = 2


def _sc_layer_body(h_hbm, src_hbm, dst_hbm, z128_hbm, iota_hbm, out_hbm,
                   src_v, rows_v, msg0, msg1, dst0, dst1, agg_sh,
                   gsem, isem):
    c = lax.axis_index("c")
    s = lax.axis_index("s")
    wid = s * _NC + c
    msgs = [msg0, msg1]
    dsts = [dst0, dst1]

    pltpu.sync_copy(src_hbm.at[pl.ds(wid * _CH, _CH)], src_v)
    pltpu.sync_copy(iota_hbm.at[s], rows_v)

    pltpu.sync_copy(z128_hbm, msg0)
    for i in range(_RC):
        pltpu.sync_copy(msg0, agg_sh.at[rows_v.at[i]])
    plsc.subcore_barrier()

    def group(g, carry):
        base = g * _NB
        copies = [
            (pltpu.async_copy(h_hbm.at[src_v.at[base + b]], msgs[b], gsem),
             pltpu.async_copy(dst_hbm.at[wid * _CH + base + b], dsts[b],
                              isem))
            for b in range(_NB)
        ]
        for b in range(_NB):
            copies[b][0].wait()
            copies[b][1].wait()
            pltpu.sync_copy(msgs[b], agg_sh.at[dsts[b].at[0]], add=True)
        return carry

    lax.fori_loop(0, _CH // _NB, group, 0)
    plsc.subcore_barrier()

    for i in range(_RC):
        pltpu.sync_copy(agg_sh.at[rows_v.at[i]], msgs[i % _NB])
        pltpu.sync_copy(
            msgs[i % _NB],
            out_hbm.at[pl.ds(c * _NP + s * _RPT + i * _K, _K)])


def _make_sc_layer():
    return pl.kernel(
        _sc_layer_body,
        out_type=[jax.ShapeDtypeStruct((_NC * _NP, _D), jnp.float32)],
        mesh=_sc_mesh(),
        scratch_types=[
            pltpu.VMEM((_CH, _K), jnp.int32),
            pltpu.VMEM((_RC, _K), jnp.int32),
            pltpu.VMEM((_K, _D), jnp.float32),
            pltpu.VMEM((_K, _D), jnp.float32),
            pltpu.VMEM((1, _K), jnp.int32),
            pltpu.VMEM((1, _K), jnp.int32),
            pltpu.VMEM_SHARED((_NP, _D), jnp.float32),
            pltpu.SemaphoreType.DMA,
            pltpu.SemaphoreType.DMA,
        ],
    )


def _sc_deg_body(dst_hbm, zdeg_hbm, ones_hbm, iota_hbm, degout_hbm,
                 dst_v, rows_v, ones_v, degbuf, deg_sh):
    c = lax.axis_index("c")
    s = lax.axis_index("s")
    wid = s * _NC + c

    pltpu.sync_copy(dst_hbm.at[pl.ds(wid * _CH, _CH)], dst_v)
    pltpu.sync_copy(iota_hbm.at[s], rows_v)
    pltpu.sync_copy(ones_hbm, ones_v)
    pltpu.sync_copy(zdeg_hbm, degbuf)
    for i in range(_RC):
        pltpu.sync_copy(degbuf, deg_sh.at[rows_v.at[i]])
    plsc.subcore_barrier()

    def step(j, carry):
        pltpu.sync_copy(ones_v, deg_sh.at[dst_v.at[j]], add=True)
        return carry

    lax.fori_loop(0, _CH, step, 0)
    plsc.subcore_barrier()

    for i in range(_RC):
        pltpu.sync_copy(deg_sh.at[rows_v.at[i]], degbuf)
        pltpu.sync_copy(
            degbuf, degout_hbm.at[pl.ds(c * _NP + s * _RPT + i * _K, _K)])


def _make_sc_deg():
    return pl.kernel(
        _sc_deg_body,
        out_type=[jax.ShapeDtypeStruct((_NC * _NP, _DW), jnp.float32)],
        mesh=_sc_mesh(),
        scratch_types=[
            pltpu.VMEM((_CH, _K), jnp.int32),
            pltpu.VMEM((_RC, _K), jnp.int32),
            pltpu.VMEM((_K, _DW), jnp.float32),
            pltpu.VMEM((_K, _DW), jnp.float32),
            pltpu.VMEM_SHARED((_NP, _DW), jnp.float32),
        ],
    )


def _tc_layer_body(relu, p_ref, deg_ref, w_ref, g_ref, bt_ref, out_ref):
    p0 = p_ref[0:_NP, :]
    p1 = p_ref[_NP:2 * _NP, :]
    deg = deg_ref[0:_NP, 0:1] + deg_ref[_NP:2 * _NP, 0:1]
    rows = lax.broadcasted_iota(jnp.int32, (_NP, 1), 0)
    inv = jnp.where(rows < _N, 1.0 / jnp.maximum(deg, 1.0), 0.0)
    agg = (p0 + p1) * inv
    z = jnp.dot(agg, w_ref[...], preferred_element_type=jnp.float32)
    mean = jnp.sum(z, axis=0, keepdims=True) * (1.0 / _N)
    var = jnp.sum(z * z, axis=0, keepdims=True) * (1.0 / _N) - mean * mean
    h = (z - mean) * lax.rsqrt(var + _EPS) * g_ref[...] + bt_ref[...]
    if relu:
        h = jnp.maximum(h, 0.0)
    out_ref[...] = h[0:_N, :]


def _make_tc_layer(relu):
    return pl.pallas_call(
        functools.partial(_tc_layer_body, relu),
        out_shape=jax.ShapeDtypeStruct((_N, _D), jnp.float32),
    )


_sc_layer = _make_sc_layer()
_sc_deg = _make_sc_deg()
_tc_layer_relu = _make_tc_layer(True)
_tc_layer_last = _make_tc_layer(False)


def kernel(x, edge_index, Ws, bs, gammas, betas):
    del bs
    src = edge_index[0]
    dst = edge_index[1]
    pad = _E_PAD - _E
    src2 = jnp.concatenate([src, jnp.zeros((pad,), jnp.int32)]).reshape(
        _E_PAD // _K, _K)
    dst_p = jnp.concatenate([dst, jnp.full((pad,), _N, jnp.int32)])
    dst2 = dst_p.reshape(_E_PAD // _K, _K)
    dst3 = dst_p.reshape(_E_PAD // _K, 1, _K)
    iota2 = jnp.arange(_NP, dtype=jnp.int32).reshape(_NS, _RC, _K)
    z128 = jnp.zeros((_K, _D), jnp.float32)
    zdeg = jnp.zeros((_K, _DW), jnp.float32)
    ones = jnp.ones((_K, _DW), jnp.float32)

    (degflat,) = _sc_deg(dst2, zdeg, ones, iota2)
    h = x
    for l in range(_L):
        (partials,) = _sc_layer(h, src2, dst3, z128, iota2)
        tc = _tc_layer_relu if l < _L - 1 else _tc_layer_last
        h = tc(partials, degflat, Ws[l],
               gammas[l].reshape(1, _D), betas[l].reshape(1, _D))
    return h
